# baseline (speedup 1.0000x reference)
import functools

import jax
import jax.numpy as jnp
import numpy as np
from jax import lax
from jax.experimental import pallas as pl
from jax.experimental.pallas import tpu as pltpu
from jax.experimental.pallas import tpu_sc as plsc

N = 10000
E = 320000
DIN = 128
DH = 128
DOUT = 40
DP = DH // 2
D3 = 64
DP3 = D3 // 2

NC = 2
NS = 16
NW = NC * NS
CH = 128
NCHUNK = 80
EPT = NCHUNK * CH
EPAD = NW * EPT
SROWS = 10112
RPT = SROWS // NS
NPAD = 10240

BLK = 2000
NBLK = N // BLK
BNC = float(1.0 / np.sqrt(1.0 + 1e-5))


_mesh = plsc.VectorSubcoreMesh(core_axis_name="c", subcore_axis_name="s")


@functools.partial(
    pl.kernel,
    out_type=jax.ShapeDtypeStruct((NW, 2, NPAD), jnp.float32),
    mesh=_mesh,
    compiler_params=pltpu.CompilerParams(needs_layout_passes=False),
    scratch_types=[
        pltpu.VMEM((NPAD,), jnp.float32),
        pltpu.VMEM((NPAD,), jnp.float32),
        pltpu.VMEM((EPT,), jnp.int32),
        pltpu.VMEM((EPT,), jnp.float32),
    ],
)
def _sc_deg(col_hbm, w_hbm, out_hbm, hw, h1, call, wall):
    c = lax.axis_index("c")
    s = lax.axis_index("s")
    wid = s * NC + c

    z = jnp.zeros((16,), jnp.float32)
    ones = jnp.ones((16,), jnp.float32)

    def zrow(i, cy):
        hw[pl.ds(i * 16, 16)] = z
        h1[pl.ds(i * 16, 16)] = z
        return cy

    lax.fori_loop(0, NPAD // 16, zrow, 0)
    pltpu.sync_copy(col_hbm.at[wid], call)
    pltpu.sync_copy(w_hbm.at[wid], wall)

    def grp(g, cy):
        cvec = call[pl.ds(g * 16, 16)]
        w16 = wall[pl.ds(g * 16, 16)]
        plsc.addupdate_scatter(hw, [cvec], w16)
        plsc.addupdate_scatter(h1, [cvec], ones)
        return cy

    lax.fori_loop(0, EPT // 16, grp, 0, unroll=2)
    pltpu.sync_copy(hw, out_hbm.at[wid, 0])
    pltpu.sync_copy(h1, out_hbm.at[wid, 1])


def _make_agg(use_w, width):
    wp2 = width // 2
    scratch = [
        pltpu.VMEM_SHARED((SROWS, width), jnp.float32),
        pltpu.VMEM((2 * NCHUNK, CH // 2), jnp.int32),
        pltpu.VMEM((2, CH), jnp.int32),
        pltpu.VMEM((2, CH, wp2), jnp.float32),
        pltpu.VMEM((CH, width), jnp.float32),
        pltpu.SemaphoreType.DMA,
        pltpu.SemaphoreType.DMA,
        pltpu.SemaphoreType.DMA,
        pltpu.SemaphoreType.DMA,
        pltpu.SemaphoreType.DMA,
        pltpu.SemaphoreType.DMA,
    ]
    if use_w:
        scratch.append(pltpu.VMEM((2, CH), jnp.float32))

    @functools.partial(
        pl.kernel,
        out_type=jax.ShapeDtypeStruct((NC, SROWS, width), jnp.float32),
        mesh=_mesh,
        compiler_params=pltpu.CompilerParams(
            needs_layout_passes=False, use_tc_tiling_on_sc=False),
        scratch_types=scratch,
    )
    def body(*refs):
        if use_w:
            (t_hbm, row_hbm, col_hbm, w_hbm, out_hbm,
             shared, cidx, ridx, raw, rows, gs0, gs1, ss0, ss1, is0, is1,
             wring) = refs
        else:
            (t_hbm, row_hbm, col_hbm, out_hbm,
             shared, cidx, ridx, raw, rows,
             gs0, gs1, ss0, ss1, is0, is1) = refs
            wring = None
        gsem = (gs0, gs1)
        isem = (is0, is1)

        c = lax.axis_index("c")
        s = lax.axis_index("s")
        wid = s * NC + c

        z = jnp.zeros((16,), jnp.float32)

        def zrow(i, cy):
            for j in range(width // 16):
                rows[i, pl.ds(j * 16, 16)] = z
            return cy

        lax.fori_loop(0, CH, zrow, 0)
        a = s * RPT
        for off in range(0, RPT, CH):
            L = min(CH, RPT - off)
            pltpu.sync_copy(rows.at[pl.ds(0, L)],
                            shared.at[pl.ds(a + off, L)])
        pltpu.sync_copy(col_hbm.at[wid], cidx)

        def idx_issue(k, b):
            pltpu.async_copy(
                row_hbm.at[wid, pl.ds(k * CH, CH)], ridx.at[b], isem[b])
            if use_w:
                pltpu.async_copy(
                    w_hbm.at[wid, pl.ds(k * CH, CH)], wring.at[b], isem[b])

        def idx_wait(k, b):
            pltpu.make_async_copy(
                row_hbm.at[wid, pl.ds(k * CH, CH)], ridx.at[b],
                isem[b]).wait()
            if use_w:
                pltpu.make_async_copy(
                    w_hbm.at[wid, pl.ds(k * CH, CH)], wring.at[b],
                    isem[b]).wait()

        def gather(b):
            pltpu.async_copy(t_hbm.at[ridx.at[b]], raw.at[b], gsem[b])

        def gather_wait(b):
            pltpu.make_async_copy(
                t_hbm.at[ridx.at[b]], raw.at[b], gsem[b]).wait()

        half = CH // 2

        def scatter(k):
            pltpu.async_copy(rows.at[pl.ds(0, half)],
                             shared.at[cidx.at[2 * k]], ss0, add=True)
            pltpu.async_copy(rows.at[pl.ds(half, half)],
                             shared.at[cidx.at[2 * k + 1]], ss1, add=True)

        def scatter_wait(k):
            pltpu.make_async_copy(rows.at[pl.ds(0, half)],
                                  shared.at[cidx.at[2 * k]], ss0).wait()
            pltpu.make_async_copy(rows.at[pl.ds(half, half)],
                                  shared.at[cidx.at[2 * k + 1]], ss1).wait()

        himask = jnp.full((16,), -65536, jnp.int32)
        sh16 = jnp.full((16,), 16, jnp.int32)

        def unpack(b):
            def ugrp(g, cy2):
                if use_w:
                    wb = wring[b, pl.ds(g * 16, 16)]
                for l in range(16):
                    e = g * 16 + l
                    if use_w:
                        wsc = wb[l]
                    for j in range(wp2 // 16):
                        v = plsc.bitcast(raw[b, e, pl.ds(j * 16, 16)],
                                         jnp.int32)
                        lo = plsc.bitcast(v << sh16, jnp.float32)
                        hi = plsc.bitcast(v & himask, jnp.float32)
                        if use_w:
                            lo = lo * wsc
                            hi = hi * wsc
                        rows[e, pl.ds(j * 16, 16)] = lo
                        rows[e, pl.ds(wp2 + j * 16, 16)] = hi
                return cy2

            lax.fori_loop(0, CH // 16, ugrp, 0)

        plsc.subcore_barrier()

        idx_issue(0, 0)
        idx_issue(1, 1)
        idx_wait(0, 0)
        gather(0)

        def pair(kk, cy):
            k0 = kk * 2
            for half in range(2):
                k = k0 + half
                b = half
                ob = 1 - half
                gather_wait(b)
                idx_wait(k + 1, ob)
                gather(ob)
                guard = kk + half

                @pl.when(guard > 0)
                def _():
                    scatter_wait(k - 1)

                unpack(b)
                idx_issue(k + 2, b)
                scatter(k)
            return cy

        lax.fori_loop(0, NCHUNK // 2 - 1, pair, 0)
        k = NCHUNK - 2
        gather_wait(0)
        idx_wait(k + 1, 1)
        gather(1)
        scatter_wait(k - 1)
        unpack(0)
        scatter(k)
        gather_wait(1)
        scatter_wait(k)
        unpack(1)
        scatter(k + 1)
        scatter_wait(k + 1)

        plsc.subcore_barrier()
        pltpu.sync_copy(shared.at[pl.ds(a, RPT)], out_hbm.at[c, pl.ds(a, RPT)])

    return body


_sc_agg_w = _make_agg(True, DH)
_sc_agg_1 = _make_agg(False, 2 * DP3)


def _deg_body(degp_ref, dis_ref):
    d = degp_ref[...]
    degw = jnp.sum(d[:, 0, :], axis=0)
    deg1 = jnp.sum(d[:, 1, :], axis=0)
    disw = jnp.where(degw > 0, lax.rsqrt(degw), 0.0)
    dis1 = jnp.where(deg1 > 0, lax.rsqrt(deg1), 0.0)
    c2 = lax.broadcasted_iota(jnp.int32, (NPAD, 2), 1)
    dis_ref[...] = jnp.where(c2 == 0, disw[:, None], dis1[:, None])


def _col_pick(arr, col):
    ci = lax.broadcasted_iota(jnp.int32, arr.shape, 1)
    return jnp.sum(jnp.where(ci == col, arr, 0.0), axis=1)


def _bf16_bits(x):
    b = lax.bitcast_convert_type(x, jnp.int32)
    r = b + jnp.int32(0x7FFF) + ((b >> 16) & 1)
    return lax.shift_right_logical(r, 16)


def _pack_rows(t):
    w2 = t.shape[1] // 2
    lo = _bf16_bits(t[:, :w2])
    hi = _bf16_bits(t[:, w2:])
    return lax.bitcast_convert_type(lo | (hi << 16), jnp.float32)


def _tc1_body(x_ref, w1_ref, dis_ref, t1_ref):
    disw = _col_pick(dis_ref[...], 0)
    t1_ref[...] = _pack_rows(jnp.dot(
        x_ref[...], w1_ref[...], preferred_element_type=jnp.float32
    ) * disw[:, None])


def _mid_body(scale_col, pp_ref, dis_ref, b_ref, g_ref, be_ref, w_ref,
              t_ref):
    p2 = pp_ref[...]
    p = p2[0] + p2[1]
    disw = _col_pick(dis_ref[...], 0)
    h = disw[:, None] * p + b_ref[...]
    h = h * BNC * g_ref[...] + be_ref[...]
    h = jnp.maximum(h, 0.0)
    dsc = _col_pick(dis_ref[...], scale_col)
    t_ref[...] = _pack_rows(jnp.dot(
        h, w_ref[...], preferred_element_type=jnp.float32) * dsc[:, None])


def _out_body(pp_ref, dis_ref, b3_ref, o_ref):
    p2 = pp_ref[...]
    p = p2[0] + p2[1]
    dis1 = _col_pick(dis_ref[...], 1)
    o_ref[...] = dis1[:, None] * p + b3_ref[...]


def _full_spec(shape):
    nd = len(shape)
    return pl.BlockSpec(shape, lambda i: (0,) * nd)


def kernel(x, edge_index, weight, W1, b1, gamma1, beta1, W2, b2, gamma2,
           beta2, W3, b3):
    f32 = jnp.float32
    row = edge_index[0]
    col = edge_index[1]
    pad = EPAD - E
    rowp = jnp.concatenate([row, jnp.zeros((pad,), jnp.int32)]).reshape(NW, EPT)
    colp = jnp.concatenate([col, jnp.full((pad,), N, jnp.int32)])
    colp3 = colp.reshape(NW, 2 * NCHUNK, CH // 2)
    colp2 = colp.reshape(NW, EPT)
    wp = jnp.concatenate([weight, jnp.zeros((pad,), f32)]).reshape(NW, EPT)

    degp = _sc_deg(colp2, wp)

    dis = pl.pallas_call(
        _deg_body,
        grid=(1,),
        in_specs=[_full_spec((NW, 2, NPAD))],
        out_specs=_full_spec((NPAD, 2)),
        out_shape=jax.ShapeDtypeStruct((NPAD, 2), f32),
    )(degp)

    t1 = pl.pallas_call(
        _tc1_body,
        grid=(NBLK,),
        in_specs=[
            pl.BlockSpec((BLK, DIN), lambda i: (i, 0)),
            _full_spec((DIN, DH)),
            pl.BlockSpec((BLK, 2), lambda i: (i, 0)),
        ],
        out_specs=pl.BlockSpec((BLK, DP), lambda i: (i, 0)),
        out_shape=jax.ShapeDtypeStruct((N, DP), f32),
    )(x, W1, dis)

    p1 = _sc_agg_w(t1, rowp, colp3, wp)

    def mid_call(body, wcols, outw):
        return pl.pallas_call(
            body,
            grid=(NBLK,),
            in_specs=[
                pl.BlockSpec((2, BLK, DH), lambda i: (0, i, 0)),
                pl.BlockSpec((BLK, 2), lambda i: (i, 0)),
                _full_spec((1, DH)),
                _full_spec((1, DH)),
                _full_spec((1, DH)),
                _full_spec((DH, wcols)),
            ],
            out_specs=pl.BlockSpec((BLK, outw), lambda i: (i, 0)),
            out_shape=jax.ShapeDtypeStruct((N, outw), f32),
        )

    t2 = mid_call(functools.partial(_mid_body, 0), DH, DP)(
        p1, dis, b1.reshape(1, DH), gamma1.reshape(1, DH),
        beta1.reshape(1, DH), W2)

    p2 = _sc_agg_w(t2, rowp, colp3, wp)

    W3pad = jnp.zeros((DH, D3), f32).at[:, :DOUT].set(W3)
    t3 = mid_call(functools.partial(_mid_body, 1), D3, DP3)(
        p2, dis, b2.reshape(1, DH), gamma2.reshape(1, DH),
        beta2.reshape(1, DH), W3pad)

    p3 = _sc_agg_1(t3, rowp, colp3)

    b3pad = jnp.zeros((1, D3), f32).at[0, :DOUT].set(b3)
    out64 = pl.pallas_call(
        _out_body,
        grid=(NBLK,),
        in_specs=[
            pl.BlockSpec((2, BLK, D3), lambda i: (0, i, 0)),
            pl.BlockSpec((BLK, 2), lambda i: (i, 0)),
            _full_spec((1, D3)),
        ],
        out_specs=pl.BlockSpec((BLK, D3), lambda i: (i, 0)),
        out_shape=jax.ShapeDtypeStruct((N, D3), f32),
    )(p3, dis, b3pad)

    return out64[:, :DOUT]


# speedup vs baseline: 1.0023x; 1.0023x over previous
import functools

import jax
import jax.numpy as jnp
import numpy as np
from jax import lax
from jax.experimental import pallas as pl
from jax.experimental.pallas import tpu as pltpu
from jax.experimental.pallas import tpu_sc as plsc

N = 10000
E = 320000
DIN = 128
DH = 128
DOUT = 40
DP = DH // 2
D3 = 64
DP3 = D3 // 2

NC = 2
NS = 16
NW = NC * NS
CH = 128
NCHUNK = 80
EPT = NCHUNK * CH
EPAD = NW * EPT
SROWS = 10112
RPT = SROWS // NS
NPAD = 10240

BLK = 2000
NBLK = N // BLK
BNC = float(1.0 / np.sqrt(1.0 + 1e-5))


_mesh = plsc.VectorSubcoreMesh(core_axis_name="c", subcore_axis_name="s")


@functools.partial(
    pl.kernel,
    out_type=jax.ShapeDtypeStruct((NW, 2, NPAD), jnp.float32),
    mesh=_mesh,
    compiler_params=pltpu.CompilerParams(needs_layout_passes=False),
    scratch_types=[
        pltpu.VMEM((NPAD,), jnp.float32),
        pltpu.VMEM((NPAD,), jnp.float32),
        pltpu.VMEM((EPT,), jnp.int32),
        pltpu.VMEM((EPT,), jnp.float32),
    ],
)
def _sc_deg(col_hbm, w_hbm, out_hbm, hw, h1, call, wall):
    c = lax.axis_index("c")
    s = lax.axis_index("s")
    wid = s * NC + c

    z = jnp.zeros((16,), jnp.float32)
    ones = jnp.ones((16,), jnp.float32)

    def zrow(i, cy):
        hw[pl.ds(i * 16, 16)] = z
        h1[pl.ds(i * 16, 16)] = z
        return cy

    lax.fori_loop(0, NPAD // 16, zrow, 0)
    pltpu.sync_copy(col_hbm.at[wid], call)
    pltpu.sync_copy(w_hbm.at[wid], wall)

    def grp(g, cy):
        cvec = call[pl.ds(g * 16, 16)]
        w16 = wall[pl.ds(g * 16, 16)]
        plsc.addupdate_scatter(hw, [cvec], w16)
        plsc.addupdate_scatter(h1, [cvec], ones)
        return cy

    lax.fori_loop(0, EPT // 16, grp, 0, unroll=2)
    pltpu.sync_copy(hw, out_hbm.at[wid, 0])
    pltpu.sync_copy(h1, out_hbm.at[wid, 1])


def _make_agg(use_w, width):
    wp2 = width // 2
    scratch = [
        pltpu.VMEM_SHARED((SROWS, width), jnp.float32),
        pltpu.VMEM((NCHUNK, CH), jnp.int32),
        pltpu.VMEM((2, CH), jnp.int32),
        pltpu.VMEM((2, CH, wp2), jnp.float32),
        pltpu.VMEM((CH, width), jnp.float32),
        pltpu.SemaphoreType.DMA,
        pltpu.SemaphoreType.DMA,
        pltpu.SemaphoreType.DMA,
        pltpu.SemaphoreType.DMA,
        pltpu.SemaphoreType.DMA,
    ]
    if use_w:
        scratch.append(pltpu.VMEM((2, CH), jnp.float32))

    @functools.partial(
        pl.kernel,
        out_type=jax.ShapeDtypeStruct((NC, SROWS, width), jnp.float32),
        mesh=_mesh,
        compiler_params=pltpu.CompilerParams(
            needs_layout_passes=False, use_tc_tiling_on_sc=False),
        scratch_types=scratch,
    )
    def body(*refs):
        if use_w:
            (t_hbm, row_hbm, col_hbm, w_hbm, out_hbm,
             shared, cidx, ridx, raw, rows, gs0, gs1, ssem, is0, is1,
             wring) = refs
        else:
            (t_hbm, row_hbm, col_hbm, out_hbm,
             shared, cidx, ridx, raw, rows,
             gs0, gs1, ssem, is0, is1) = refs
            wring = None
        gsem = (gs0, gs1)
        isem = (is0, is1)

        c = lax.axis_index("c")
        s = lax.axis_index("s")
        wid = s * NC + c

        z = jnp.zeros((16,), jnp.float32)

        def zrow(i, cy):
            for j in range(width // 16):
                rows[i, pl.ds(j * 16, 16)] = z
            return cy

        lax.fori_loop(0, CH, zrow, 0)
        a = s * RPT
        for off in range(0, RPT, CH):
            L = min(CH, RPT - off)
            pltpu.sync_copy(rows.at[pl.ds(0, L)],
                            shared.at[pl.ds(a + off, L)])
        pltpu.sync_copy(col_hbm.at[wid], cidx)

        def idx_issue(k, b):
            pltpu.async_copy(
                row_hbm.at[wid, pl.ds(k * CH, CH)], ridx.at[b], isem[b])
            if use_w:
                pltpu.async_copy(
                    w_hbm.at[wid, pl.ds(k * CH, CH)], wring.at[b], isem[b])

        def idx_wait(k, b):
            pltpu.make_async_copy(
                row_hbm.at[wid, pl.ds(k * CH, CH)], ridx.at[b],
                isem[b]).wait()
            if use_w:
                pltpu.make_async_copy(
                    w_hbm.at[wid, pl.ds(k * CH, CH)], wring.at[b],
                    isem[b]).wait()

        def gather(b):
            pltpu.async_copy(t_hbm.at[ridx.at[b]], raw.at[b], gsem[b])

        def gather_wait(b):
            pltpu.make_async_copy(
                t_hbm.at[ridx.at[b]], raw.at[b], gsem[b]).wait()

        def scatter(k):
            pltpu.async_copy(
                rows, shared.at[cidx.at[k]], ssem, add=True)

        def scatter_wait(k):
            pltpu.make_async_copy(
                rows, shared.at[cidx.at[k]], ssem).wait()

        himask = jnp.full((16,), -65536, jnp.int32)
        sh16 = jnp.full((16,), 16, jnp.int32)

        def unpack(b):
            def ugrp(g, cy2):
                if use_w:
                    wb = wring[b, pl.ds(g * 16, 16)]
                for l in range(16):
                    e = g * 16 + l
                    if use_w:
                        wsc = wb[l]
                    for j in range(wp2 // 16):
                        v = plsc.bitcast(raw[b, e, pl.ds(j * 16, 16)],
                                         jnp.int32)
                        lo = plsc.bitcast(v << sh16, jnp.float32)
                        hi = plsc.bitcast(v & himask, jnp.float32)
                        if use_w:
                            lo = lo * wsc
                            hi = hi * wsc
                        rows[e, pl.ds(j * 16, 16)] = lo
                        rows[e, pl.ds(wp2 + j * 16, 16)] = hi
                return cy2

            lax.fori_loop(0, CH // 16, ugrp, 0)

        plsc.subcore_barrier()

        idx_issue(0, 0)
        idx_issue(1, 1)
        idx_wait(0, 0)
        gather(0)

        def pair(kk, cy):
            k0 = kk * 2
            for half in range(2):
                k = k0 + half
                b = half
                ob = 1 - half
                gather_wait(b)
                idx_wait(k + 1, ob)
                gather(ob)
                guard = kk + half

                @pl.when(guard > 0)
                def _():
                    scatter_wait(k - 1)

                unpack(b)
                idx_issue(k + 2, b)
                scatter(k)
            return cy

        lax.fori_loop(0, NCHUNK // 2 - 1, pair, 0)
        k = NCHUNK - 2
        gather_wait(0)
        idx_wait(k + 1, 1)
        gather(1)
        scatter_wait(k - 1)
        unpack(0)
        scatter(k)
        gather_wait(1)
        scatter_wait(k)
        unpack(1)
        scatter(k + 1)
        scatter_wait(k + 1)

        plsc.subcore_barrier()
        pltpu.sync_copy(shared.at[pl.ds(a, RPT)], out_hbm.at[c, pl.ds(a, RPT)])

    return body


_sc_agg_w = _make_agg(True, DH)
_sc_agg_1 = _make_agg(False, 2 * DP3)


def _deg_body(degp_ref, dis_ref):
    d = degp_ref[...]
    degw = jnp.sum(d[:, 0, :], axis=0)
    deg1 = jnp.sum(d[:, 1, :], axis=0)
    disw = jnp.where(degw > 0, lax.rsqrt(degw), 0.0)
    dis1 = jnp.where(deg1 > 0, lax.rsqrt(deg1), 0.0)
    c2 = lax.broadcasted_iota(jnp.int32, (NPAD, 2), 1)
    dis_ref[...] = jnp.where(c2 == 0, disw[:, None], dis1[:, None])


def _col_pick(arr, col):
    ci = lax.broadcasted_iota(jnp.int32, arr.shape, 1)
    return jnp.sum(jnp.where(ci == col, arr, 0.0), axis=1)


def _bf16_bits(x):
    b = lax.bitcast_convert_type(x, jnp.int32)
    r = b + jnp.int32(0x7FFF) + ((b >> 16) & 1)
    return lax.shift_right_logical(r, 16)


def _pack_rows(t):
    w2 = t.shape[1] // 2
    lo = _bf16_bits(t[:, :w2])
    hi = _bf16_bits(t[:, w2:])
    return lax.bitcast_convert_type(lo | (hi << 16), jnp.float32)


def _tc1_body(x_ref, w1_ref, dis_ref, t1_ref):
    disw = _col_pick(dis_ref[...], 0)
    t1_ref[...] = _pack_rows(jnp.dot(
        x_ref[...], w1_ref[...], preferred_element_type=jnp.float32
    ) * disw[:, None])


def _mid_body(scale_col, pp_ref, dis_ref, b_ref, g_ref, be_ref, w_ref,
              t_ref):
    p2 = pp_ref[...]
    p = p2[0] + p2[1]
    disw = _col_pick(dis_ref[...], 0)
    h = disw[:, None] * p + b_ref[...]
    h = h * BNC * g_ref[...] + be_ref[...]
    h = jnp.maximum(h, 0.0)
    dsc = _col_pick(dis_ref[...], scale_col)
    t_ref[...] = _pack_rows(jnp.dot(
        h, w_ref[...], preferred_element_type=jnp.float32) * dsc[:, None])


def _out_body(pp_ref, dis_ref, b3_ref, o_ref):
    p2 = pp_ref[...]
    p = p2[0] + p2[1]
    dis1 = _col_pick(dis_ref[...], 1)
    o_ref[...] = dis1[:, None] * p + b3_ref[...]


def _full_spec(shape):
    nd = len(shape)
    return pl.BlockSpec(shape, lambda i: (0,) * nd)


def kernel(x, edge_index, weight, W1, b1, gamma1, beta1, W2, b2, gamma2,
           beta2, W3, b3):
    f32 = jnp.float32
    row = edge_index[0]
    col = edge_index[1]
    pad = EPAD - E
    rowp = jnp.concatenate([row, jnp.zeros((pad,), jnp.int32)]).reshape(NW, EPT)
    colp = jnp.concatenate([col, jnp.full((pad,), N, jnp.int32)])
    colp3 = colp.reshape(NW, NCHUNK, CH)
    colp2 = colp.reshape(NW, EPT)
    wp = jnp.concatenate([weight, jnp.zeros((pad,), f32)]).reshape(NW, EPT)

    degp = _sc_deg(colp2, wp)

    dis = pl.pallas_call(
        _deg_body,
        grid=(1,),
        in_specs=[_full_spec((NW, 2, NPAD))],
        out_specs=_full_spec((NPAD, 2)),
        out_shape=jax.ShapeDtypeStruct((NPAD, 2), f32),
    )(degp)

    t1 = pl.pallas_call(
        _tc1_body,
        grid=(NBLK,),
        in_specs=[
            pl.BlockSpec((BLK, DIN), lambda i: (i, 0)),
            _full_spec((DIN, DH)),
            pl.BlockSpec((BLK, 2), lambda i: (i, 0)),
        ],
        out_specs=pl.BlockSpec((BLK, DP), lambda i: (i, 0)),
        out_shape=jax.ShapeDtypeStruct((N, DP), f32),
    )(x, W1, dis)

    p1 = _sc_agg_w(t1, rowp, colp3, wp)

    def mid_call(body, wcols, outw):
        return pl.pallas_call(
            body,
            grid=(NBLK,),
            in_specs=[
                pl.BlockSpec((2, BLK, DH), lambda i: (0, i, 0)),
                pl.BlockSpec((BLK, 2), lambda i: (i, 0)),
                _full_spec((1, DH)),
                _full_spec((1, DH)),
                _full_spec((1, DH)),
                _full_spec((DH, wcols)),
            ],
            out_specs=pl.BlockSpec((BLK, outw), lambda i: (i, 0)),
            out_shape=jax.ShapeDtypeStruct((N, outw), f32),
        )

    t2 = mid_call(functools.partial(_mid_body, 0), DH, DP)(
        p1, dis, b1.reshape(1, DH), gamma1.reshape(1, DH),
        beta1.reshape(1, DH), W2)

    p2 = _sc_agg_w(t2, rowp, colp3, wp)

    W3pad = jnp.zeros((DH, D3), f32).at[:, :DOUT].set(W3)
    t3 = mid_call(functools.partial(_mid_body, 1), D3, DP3)(
        p2, dis, b2.reshape(1, DH), gamma2.reshape(1, DH),
        beta2.reshape(1, DH), W3pad)

    p3 = _sc_agg_1(t3, rowp, colp3)

    b3pad = jnp.zeros((1, D3), f32).at[0, :DOUT].set(b3)
    out64 = pl.pallas_call(
        _out_body,
        grid=(NBLK,),
        in_specs=[
            pl.BlockSpec((2, BLK, D3), lambda i: (0, i, 0)),
            pl.BlockSpec((BLK, 2), lambda i: (i, 0)),
            _full_spec((1, D3)),
        ],
        out_specs=pl.BlockSpec((BLK, D3), lambda i: (i, 0)),
        out_shape=jax.ShapeDtypeStruct((N, D3), f32),
    )(p3, dis, b3pad)

    return out64[:, :DOUT]
